# baseline (speedup 1.0000x reference)
import functools

import jax
import jax.numpy as jnp
from jax import lax
from jax.experimental import pallas as pl
from jax.experimental.pallas import tpu as pltpu
from jax.experimental.pallas import tpu_sc as plsc

N = 10000
E = 320000
D_IN = 128
HID = 16
D_OUT = 128

NP = 10240
ROWS_PER_TILE = NP // 16
NW = 32
CHUNK = 128
CHUNKS_PER_W = 82
EPW = CHUNK * CHUNKS_PER_W
EP = NW * EPW

_mesh = plsc.VectorSubcoreMesh(core_axis_name="c", subcore_axis_name="s")


def _zero_fill(ref, rows):
    z = jnp.zeros((16,), jnp.float32)

    def body(i, _):
        ref[i] = z
        return 0

    lax.fori_loop(0, rows, body, 0)


@functools.partial(
    pl.kernel,
    mesh=_mesh,
    compiler_params=pltpu.CompilerParams(use_tc_tiling_on_sc=False),
    out_type=jax.ShapeDtypeStruct((2, NP, 16), jnp.float32),
    scratch_types=[
        pltpu.VMEM((CHUNKS_PER_W, CHUNK), jnp.int32),
        pltpu.VMEM((CHUNK, 16), jnp.float32),
        pltpu.VMEM((ROWS_PER_TILE, 16), jnp.float32),
        pltpu.VMEM_SHARED((NP, 16), jnp.float32),
    ],
)
def _sc_count(dst_hbm, out_hbm, dst_v, ones_v, zslab_v, acc):
    c = lax.axis_index("c")
    s = lax.axis_index("s")
    wid = s * 2 + c

    pltpu.sync_copy(dst_hbm.at[wid], dst_v)

    one = jnp.full((16,), 1.0, jnp.float32)

    def fill_ones(i, _):
        ones_v[i] = one
        return 0

    lax.fori_loop(0, CHUNK, fill_ones, 0)

    _zero_fill(zslab_v, ROWS_PER_TILE)
    pltpu.sync_copy(zslab_v, acc.at[pl.ds(s * ROWS_PER_TILE, ROWS_PER_TILE)])
    plsc.subcore_barrier()

    def body(j, _):
        pltpu.sync_copy(ones_v, acc.at[dst_v.at[j]], add=True)
        return 0

    lax.fori_loop(0, CHUNKS_PER_W, body, 0)

    plsc.subcore_barrier()
    sl = pl.ds(s * ROWS_PER_TILE, ROWS_PER_TILE)
    pltpu.sync_copy(acc.at[sl], out_hbm.at[c, sl])


@functools.partial(
    pl.kernel,
    mesh=_mesh,
    compiler_params=pltpu.CompilerParams(use_tc_tiling_on_sc=False),
    out_type=jax.ShapeDtypeStruct((2, NP, 16), jnp.float32),
    scratch_types=[
        pltpu.VMEM((CHUNKS_PER_W, CHUNK), jnp.int32),
        pltpu.VMEM((CHUNKS_PER_W, CHUNK), jnp.int32),
        pltpu.VMEM((CHUNK, 16), jnp.float32),
        pltpu.VMEM((CHUNK, 16), jnp.float32),
        pltpu.VMEM((ROWS_PER_TILE, 16), jnp.float32),
        pltpu.VMEM_SHARED((NP, 16), jnp.float32),
        pltpu.VMEM_SHARED((NP, 16), jnp.float32),
        pltpu.SemaphoreType.DMA,
        pltpu.SemaphoreType.DMA,
    ],
)
def _sc_prop(g_hbm, src_hbm, dst_hbm, out_hbm,
             src_v, dst_v, row0, row1, zslab_v, acc, gtab, sem0, sem1):
    c = lax.axis_index("c")
    s = lax.axis_index("s")
    wid = s * 2 + c

    pltpu.sync_copy(src_hbm.at[wid], src_v)
    pltpu.sync_copy(dst_hbm.at[wid], dst_v)
    _zero_fill(zslab_v, ROWS_PER_TILE)
    stage = pl.ds(s * ROWS_PER_TILE, ROWS_PER_TILE)
    pltpu.sync_copy(zslab_v, acc.at[stage])
    pltpu.sync_copy(g_hbm.at[stage], gtab.at[stage])
    plsc.subcore_barrier()

    bufs = (row0, row1)
    sems = (sem0, sem1)

    pltpu.async_copy(gtab.at[src_v.at[0]], row0, sem0)
    pltpu.async_copy(gtab.at[src_v.at[1]], row1, sem1)

    def outer(t, _):
        j = t * 2
        for b in range(2):
            jj = j + b
            pltpu.make_async_copy(gtab.at[src_v.at[jj]], bufs[b], sems[b]).wait()
            pltpu.sync_copy(bufs[b], acc.at[dst_v.at[jj]], add=True)

            @pl.when(jj + 2 < CHUNKS_PER_W)
            def _start():
                pltpu.async_copy(gtab.at[src_v.at[jj + 2]], bufs[b], sems[b])
        return 0

    lax.fori_loop(0, CHUNKS_PER_W // 2, outer, 0)

    plsc.subcore_barrier()
    sl = pl.ds(s * ROWS_PER_TILE, ROWS_PER_TILE)
    pltpu.sync_copy(acc.at[sl], out_hbm.at[c, sl])


def _tc_mm1(x_ref, w_ref, o_ref):
    o_ref[:] = jnp.dot(x_ref[:], w_ref[:], preferred_element_type=jnp.float32)


def _tc_scale1(h1_ref, cnt_ref, og_ref, od_ref):
    deg = cnt_ref[0, :, 0:1] + cnt_ref[1, :, 0:1]
    dinv = lax.rsqrt(deg)
    od_ref[:] = jnp.broadcast_to(dinv, (NP, 16))
    og_ref[:] = h1_ref[:] * dinv


def _tc_scale2(sp_ref, dinv_ref, b1_ref, o_ref):
    s = sp_ref[0] + sp_ref[1]
    h = jnp.maximum(dinv_ref[:] * s + b1_ref[:], 0.0)
    o_ref[:] = dinv_ref[:] * h


def _tc_mm2(sp_ref, dinv_ref, w_ref, b_ref, o_ref):
    a = dinv_ref[:] * (sp_ref[0] + sp_ref[1])
    o_ref[:] = jnp.dot(a, w_ref[:], preferred_element_type=jnp.float32) + b_ref[:]


def kernel(x, edge_index, W1, b1, W2, b2):
    f32 = jnp.float32

    loop = jnp.arange(N, dtype=jnp.int32)
    pad = jnp.full((EP - E - N,), N, dtype=jnp.int32)
    src_ext = jnp.concatenate([edge_index[0], loop, pad]).reshape(NW, CHUNKS_PER_W, CHUNK)
    dst_ext = jnp.concatenate([edge_index[1], loop, pad]).reshape(NW, CHUNKS_PER_W, CHUNK)
    x_pad = jnp.zeros((NP, D_IN), f32).at[:N].set(x)

    counts = _sc_count(dst_ext)
    h1 = pl.pallas_call(
        _tc_mm1,
        out_shape=jax.ShapeDtypeStruct((NP, HID), f32),
    )(x_pad, W1)

    g1, dinv16 = pl.pallas_call(
        _tc_scale1,
        out_shape=(
            jax.ShapeDtypeStruct((NP, HID), f32),
            jax.ShapeDtypeStruct((NP, HID), f32),
        ),
    )(h1, counts)

    s1 = _sc_prop(g1, src_ext, dst_ext)

    g2 = pl.pallas_call(
        _tc_scale2,
        out_shape=jax.ShapeDtypeStruct((NP, HID), f32),
    )(s1, dinv16, b1.reshape(1, HID))

    s2 = _sc_prop(g2, src_ext, dst_ext)

    out = pl.pallas_call(
        _tc_mm2,
        out_shape=jax.ShapeDtypeStruct((NP, D_OUT), f32),
    )(s2, dinv16, W2, b2.reshape(1, D_OUT))

    return out[:N]


# speedup vs baseline: 1.3055x; 1.3055x over previous
import functools

import jax
import jax.numpy as jnp
from jax import lax
from jax.experimental import pallas as pl
from jax.experimental.pallas import tpu as pltpu
from jax.experimental.pallas import tpu_sc as plsc

N = 10000
E = 320000
D_IN = 128
HID = 16
D_OUT = 128

RPT = N // 16
CHUNK = 128
NCHUNKS = E // CHUNK
BASE_CH = NCHUNKS // 32
EXTRA = NCHUNKS - 32 * BASE_CH
MAXCH = BASE_CH + 1

_mesh = plsc.VectorSubcoreMesh(core_axis_name="c", subcore_axis_name="s")
_params = pltpu.CompilerParams(use_tc_tiling_on_sc=False)


def _worker_chunks(w):
    return jnp.where(w < EXTRA, BASE_CH + 1, BASE_CH)


def _load_indices(idx_hbm, idx_v, w):
    pltpu.sync_copy(idx_hbm.at[pl.ds(w * BASE_CH, BASE_CH)],
                    idx_v.at[pl.ds(0, BASE_CH)])

    @pl.when(w < EXTRA)
    def _tail():
        pltpu.sync_copy(idx_hbm.at[pl.ds(32 * BASE_CH + w, 1)],
                        idx_v.at[pl.ds(BASE_CH, 1)])


def _fill_rows(ref, rows, vec):
    def body(i, _):
        ref[i] = vec
        return 0

    lax.fori_loop(0, rows, body, 0)


def _newton_rsqrt(d):
    i = lax.bitcast_convert_type(d, jnp.int32)
    i = jnp.int32(0x5F3759DF) - lax.shift_right_logical(i, 1)
    y = lax.bitcast_convert_type(i, jnp.float32)
    half_d = 0.5 * d
    for _ in range(3):
        y = y * (1.5 - half_d * y * y)
    return y


def _edge_pass(gtab, acc, src_v, dst_v, row0, row1, sem0, sem1, nch):
    bufs = (row0, row1)
    sems = (sem0, sem1)

    pltpu.async_copy(gtab.at[src_v.at[0]], row0, sem0)
    pltpu.async_copy(gtab.at[src_v.at[1]], row1, sem1)

    def outer(t, _):
        j = t * 2
        for b in range(2):
            jj = j + b
            pltpu.make_async_copy(gtab.at[src_v.at[jj]], bufs[b], sems[b]).wait()
            pltpu.sync_copy(bufs[b], acc.at[dst_v.at[jj]], add=True)

            @pl.when(jj + 2 < nch)
            def _refill():
                pltpu.async_copy(gtab.at[src_v.at[jj + 2]], bufs[b], sems[b])
        return 0

    lax.fori_loop(0, nch // 2, outer, 0)

    @pl.when(nch % 2 == 1)
    def _last():
        jj = nch - 1
        pltpu.make_async_copy(gtab.at[src_v.at[jj]], row0, sem0).wait()
        pltpu.sync_copy(row0, acc.at[dst_v.at[jj]], add=True)


@functools.partial(
    pl.kernel,
    mesh=_mesh,
    compiler_params=_params,
    out_type=jax.ShapeDtypeStruct((2, N, 16), jnp.float32),
    scratch_types=[
        pltpu.VMEM((MAXCH, CHUNK), jnp.int32),
        pltpu.VMEM((CHUNK, 16), jnp.float32),
        pltpu.VMEM((RPT, 16), jnp.float32),
        pltpu.VMEM_SHARED((N, 16), jnp.float32),
    ],
)
def _sc_count(dst_hbm, out_hbm, dst_v, ones_v, slab_v, acc):
    c = lax.axis_index("c")
    s = lax.axis_index("s")
    wid = s * 2 + c

    _load_indices(dst_hbm, dst_v, wid)

    one = jnp.full((16,), 1.0, jnp.float32)
    _fill_rows(ones_v, CHUNK, one)
    init = jnp.where(c == 0, 1.0, 0.0) * one
    _fill_rows(slab_v, RPT, init)
    sl = pl.ds(s * RPT, RPT)
    pltpu.sync_copy(slab_v, acc.at[sl])
    plsc.subcore_barrier()

    nch = _worker_chunks(wid)

    def body(j, _):
        pltpu.sync_copy(ones_v, acc.at[dst_v.at[j]], add=True)
        return 0

    lax.fori_loop(0, nch, body, 0)

    plsc.subcore_barrier()
    pltpu.sync_copy(acc.at[sl], out_hbm.at[c, sl])


@functools.partial(
    pl.kernel,
    mesh=_mesh,
    compiler_params=_params,
    out_type=(
        jax.ShapeDtypeStruct((2, N, 16), jnp.float32),
        jax.ShapeDtypeStruct((N, 16), jnp.float32),
    ),
    scratch_types=[
        pltpu.VMEM((MAXCH, CHUNK), jnp.int32),
        pltpu.VMEM((MAXCH, CHUNK), jnp.int32),
        pltpu.VMEM((CHUNK, 16), jnp.float32),
        pltpu.VMEM((CHUNK, 16), jnp.float32),
        pltpu.VMEM((RPT, 16), jnp.float32),
        pltpu.VMEM((RPT, 16), jnp.float32),
        pltpu.VMEM((RPT, 16), jnp.float32),
        pltpu.VMEM((RPT, 16), jnp.float32),
        pltpu.VMEM((RPT, 16), jnp.float32),
        pltpu.VMEM_SHARED((N, 16), jnp.float32),
        pltpu.VMEM_SHARED((N, 16), jnp.float32),
        pltpu.SemaphoreType.DMA,
        pltpu.SemaphoreType.DMA,
    ],
)
def _sc_prop1(h1_hbm, cnt_hbm, src_hbm, dst_hbm, s1_hbm, dinv_hbm,
              src_v, dst_v, row0, row1, p0_v, p1_v, h1_v, g1_v, dinv_v,
              acc, gtab, sem0, sem1):
    c = lax.axis_index("c")
    s = lax.axis_index("s")
    wid = s * 2 + c

    _load_indices(src_hbm, src_v, wid)
    _load_indices(dst_hbm, dst_v, wid)

    sl = pl.ds(s * RPT, RPT)
    pltpu.sync_copy(cnt_hbm.at[0, sl], p0_v)
    pltpu.sync_copy(cnt_hbm.at[1, sl], p1_v)
    pltpu.sync_copy(h1_hbm.at[sl], h1_v)

    def scale(i, _):
        deg = p0_v[i] + p1_v[i]
        dv = _newton_rsqrt(deg)
        dinv_v[i] = dv
        g1_v[i] = dv * h1_v[i]
        return 0

    lax.fori_loop(0, RPT, scale, 0)

    pltpu.sync_copy(g1_v, gtab.at[sl])
    pltpu.sync_copy(dinv_v, dinv_hbm.at[sl])

    zero = jnp.zeros((16,), jnp.float32)
    _fill_rows(p0_v, RPT, zero)

    @pl.when(c == 0)
    def _ia():
        pltpu.sync_copy(g1_v, acc.at[sl])

    @pl.when(c != 0)
    def _ib():
        pltpu.sync_copy(p0_v, acc.at[sl])

    plsc.subcore_barrier()
    _edge_pass(gtab, acc, src_v, dst_v, row0, row1, sem0, sem1,
               _worker_chunks(wid))
    plsc.subcore_barrier()
    pltpu.sync_copy(acc.at[sl], s1_hbm.at[c, sl])


@functools.partial(
    pl.kernel,
    mesh=_mesh,
    compiler_params=_params,
    out_type=jax.ShapeDtypeStruct((2, N, 16), jnp.float32),
    scratch_types=[
        pltpu.VMEM((MAXCH, CHUNK), jnp.int32),
        pltpu.VMEM((MAXCH, CHUNK), jnp.int32),
        pltpu.VMEM((CHUNK, 16), jnp.float32),
        pltpu.VMEM((CHUNK, 16), jnp.float32),
        pltpu.VMEM((RPT, 16), jnp.float32),
        pltpu.VMEM((RPT, 16), jnp.float32),
        pltpu.VMEM((RPT, 16), jnp.float32),
        pltpu.VMEM((RPT, 16), jnp.float32),
        pltpu.VMEM((16,), jnp.float32),
        pltpu.VMEM_SHARED((N, 16), jnp.float32),
        pltpu.VMEM_SHARED((N, 16), jnp.float32),
        pltpu.SemaphoreType.DMA,
        pltpu.SemaphoreType.DMA,
    ],
)
def _sc_prop2(s1_hbm, dinv_hbm, b1_hbm, src_hbm, dst_hbm, s2_hbm,
              src_v, dst_v, row0, row1, p0_v, p1_v, dinv_v, g2_v, b1_v,
              acc, gtab, sem0, sem1):
    c = lax.axis_index("c")
    s = lax.axis_index("s")
    wid = s * 2 + c

    _load_indices(src_hbm, src_v, wid)
    _load_indices(dst_hbm, dst_v, wid)

    sl = pl.ds(s * RPT, RPT)
    pltpu.sync_copy(s1_hbm.at[0, sl], p0_v)
    pltpu.sync_copy(s1_hbm.at[1, sl], p1_v)
    pltpu.sync_copy(dinv_hbm.at[sl], dinv_v)
    pltpu.sync_copy(b1_hbm, b1_v)

    def scale(i, _):
        dv = dinv_v[i]
        a1 = dv * (p0_v[i] + p1_v[i]) + b1_v[...]
        g2_v[i] = dv * jnp.maximum(a1, 0.0)
        return 0

    lax.fori_loop(0, RPT, scale, 0)

    pltpu.sync_copy(g2_v, gtab.at[sl])

    zero = jnp.zeros((16,), jnp.float32)
    _fill_rows(p0_v, RPT, zero)

    @pl.when(c == 0)
    def _ia():
        pltpu.sync_copy(g2_v, acc.at[sl])

    @pl.when(c != 0)
    def _ib():
        pltpu.sync_copy(p0_v, acc.at[sl])

    plsc.subcore_barrier()
    _edge_pass(gtab, acc, src_v, dst_v, row0, row1, sem0, sem1,
               _worker_chunks(wid))
    plsc.subcore_barrier()
    pltpu.sync_copy(acc.at[sl], s2_hbm.at[c, sl])


def _tc_mm1(x_ref, w_ref, o_ref):
    o_ref[:] = jnp.dot(x_ref[:], w_ref[:], preferred_element_type=jnp.float32)


def _tc_mm2(sp_ref, dinv_ref, w_ref, b_ref, o_ref):
    a = dinv_ref[:] * (sp_ref[0] + sp_ref[1])
    o_ref[:] = jnp.dot(a, w_ref[:], preferred_element_type=jnp.float32) + b_ref[:]


def kernel(x, edge_index, W1, b1, W2, b2):
    f32 = jnp.float32

    src2d = edge_index[0].reshape(NCHUNKS, CHUNK)
    dst2d = edge_index[1].reshape(NCHUNKS, CHUNK)

    counts = _sc_count(dst2d)
    h1 = pl.pallas_call(
        _tc_mm1,
        out_shape=jax.ShapeDtypeStruct((N, HID), f32),
    )(x, W1)

    s1, dinv16 = _sc_prop1(h1, counts, src2d, dst2d)
    s2 = _sc_prop2(s1, dinv16, b1, src2d, dst2d)

    out = pl.pallas_call(
        _tc_mm2,
        out_shape=jax.ShapeDtypeStruct((N, D_OUT), f32),
    )(s2, dinv16, W2, b2.reshape(1, D_OUT))

    return out


# speedup vs baseline: 1.5313x; 1.1729x over previous
import functools

import jax
import jax.numpy as jnp
from jax import lax
from jax.experimental import pallas as pl
from jax.experimental.pallas import tpu as pltpu
from jax.experimental.pallas import tpu_sc as plsc

N = 10000
E = 320000
D_IN = 128
HID = 16
D_OUT = 128

RPT = N // 16
EB = 1000
NB = 10
EROWS = E // EB

_mesh = plsc.VectorSubcoreMesh(core_axis_name="c", subcore_axis_name="s")
_params = pltpu.CompilerParams(use_tc_tiling_on_sc=False)


def _fill_rows(ref, rows, vec, unroll=8):
    def body(i, _):
        for u in range(unroll):
            ref[i * unroll + u] = vec
        return 0

    lax.fori_loop(0, rows // unroll, body, 0)
    for u in range(rows - rows % unroll, rows):
        ref[u] = vec


def _newton_rsqrt(d):
    i = lax.bitcast_convert_type(d, jnp.int32)
    i = jnp.int32(0x5F3759DF) - lax.shift_right_logical(i, 1)
    y = lax.bitcast_convert_type(i, jnp.float32)
    half_d = 0.5 * d
    for _ in range(3):
        y = y * (1.5 - half_d * y * y)
    return y


def _edge_pass(gtab, acc, src_v, dst_v, row0, row1, sem0, sem1):
    bufs = (row0, row1)
    sems = (sem0, sem1)

    pltpu.async_copy(gtab.at[src_v.at[0]], row0, sem0)
    pltpu.async_copy(gtab.at[src_v.at[1]], row1, sem1)
    for j in range(NB):
        b = j % 2
        pltpu.make_async_copy(gtab.at[src_v.at[j]], bufs[b], sems[b]).wait()
        pltpu.sync_copy(bufs[b], acc.at[dst_v.at[j]], add=True)
        if j + 2 < NB:
            pltpu.async_copy(gtab.at[src_v.at[j + 2]], bufs[b], sems[b])


@functools.partial(
    pl.kernel,
    mesh=_mesh,
    compiler_params=_params,
    out_type=jax.ShapeDtypeStruct((2, N, 16), jnp.float32),
    scratch_types=[
        pltpu.VMEM((NB, EB), jnp.int32),
        pltpu.VMEM((EB, 16), jnp.float32),
        pltpu.VMEM((RPT, 16), jnp.float32),
        pltpu.VMEM_SHARED((N, 16), jnp.float32),
    ],
)
def _sc_count(dst_hbm, out_hbm, dst_v, ones_v, slab_v, acc):
    c = lax.axis_index("c")
    s = lax.axis_index("s")
    wid = s * 2 + c

    pltpu.sync_copy(dst_hbm.at[pl.ds(wid * NB, NB)], dst_v)

    one = jnp.full((16,), 1.0, jnp.float32)
    _fill_rows(ones_v, EB, one)
    init = jnp.where(c == 0, 1.0, 0.0) * one
    _fill_rows(slab_v, RPT, init)
    sl = pl.ds(s * RPT, RPT)
    pltpu.sync_copy(slab_v, acc.at[sl])
    plsc.subcore_barrier()

    for j in range(NB):
        pltpu.sync_copy(ones_v, acc.at[dst_v.at[j]], add=True)

    plsc.subcore_barrier()
    pltpu.sync_copy(acc.at[sl], out_hbm.at[c, sl])


@functools.partial(
    pl.kernel,
    mesh=_mesh,
    compiler_params=_params,
    out_type=(
        jax.ShapeDtypeStruct((2, N, 16), jnp.float32),
        jax.ShapeDtypeStruct((N, 16), jnp.float32),
    ),
    scratch_types=[
        pltpu.VMEM((NB, EB), jnp.int32),
        pltpu.VMEM((NB, EB), jnp.int32),
        pltpu.VMEM((EB, 16), jnp.float32),
        pltpu.VMEM((EB, 16), jnp.float32),
        pltpu.VMEM((RPT, 16), jnp.float32),
        pltpu.VMEM((RPT, 16), jnp.float32),
        pltpu.VMEM((RPT, 16), jnp.float32),
        pltpu.VMEM((RPT, 16), jnp.float32),
        pltpu.VMEM_SHARED((N, 16), jnp.float32),
        pltpu.VMEM_SHARED((N, 16), jnp.float32),
        pltpu.SemaphoreType.DMA,
        pltpu.SemaphoreType.DMA,
    ],
)
def _sc_prop1(h1_hbm, cnt_hbm, src_hbm, dst_hbm, s1_hbm, dinv_hbm,
              src_v, dst_v, row0, row1, p0_v, p1_v, hd_v, g1_v,
              acc, gtab, sem0, sem1):
    c = lax.axis_index("c")
    s = lax.axis_index("s")
    wid = s * 2 + c

    pltpu.sync_copy(src_hbm.at[pl.ds(wid * NB, NB)], src_v)
    pltpu.sync_copy(dst_hbm.at[pl.ds(wid * NB, NB)], dst_v)

    sl = pl.ds(s * RPT, RPT)
    pltpu.sync_copy(cnt_hbm.at[0, sl], p0_v)
    pltpu.sync_copy(cnt_hbm.at[1, sl], p1_v)
    pltpu.sync_copy(h1_hbm.at[sl], hd_v)

    def scale(i, _):
        deg = p0_v[i] + p1_v[i]
        dv = _newton_rsqrt(deg)
        g1_v[i] = dv * hd_v[i]
        hd_v[i] = dv
        return 0

    lax.fori_loop(0, RPT, scale, 0)

    pltpu.sync_copy(g1_v, gtab.at[sl])
    pltpu.sync_copy(hd_v, dinv_hbm.at[sl])

    zero = jnp.zeros((16,), jnp.float32)
    _fill_rows(p0_v, RPT, zero)

    @pl.when(c == 0)
    def _ia():
        pltpu.sync_copy(g1_v, acc.at[sl])

    @pl.when(c != 0)
    def _ib():
        pltpu.sync_copy(p0_v, acc.at[sl])

    plsc.subcore_barrier()
    _edge_pass(gtab, acc, src_v, dst_v, row0, row1, sem0, sem1)
    plsc.subcore_barrier()
    pltpu.sync_copy(acc.at[sl], s1_hbm.at[c, sl])


@functools.partial(
    pl.kernel,
    mesh=_mesh,
    compiler_params=_params,
    out_type=jax.ShapeDtypeStruct((2, N, 16), jnp.float32),
    scratch_types=[
        pltpu.VMEM((NB, EB), jnp.int32),
        pltpu.VMEM((NB, EB), jnp.int32),
        pltpu.VMEM((EB, 16), jnp.float32),
        pltpu.VMEM((EB, 16), jnp.float32),
        pltpu.VMEM((RPT, 16), jnp.float32),
        pltpu.VMEM((RPT, 16), jnp.float32),
        pltpu.VMEM((RPT, 16), jnp.float32),
        pltpu.VMEM((RPT, 16), jnp.float32),
        pltpu.VMEM((16,), jnp.float32),
        pltpu.VMEM_SHARED((N, 16), jnp.float32),
        pltpu.VMEM_SHARED((N, 16), jnp.float32),
        pltpu.SemaphoreType.DMA,
        pltpu.SemaphoreType.DMA,
    ],
)
def _sc_prop2(s1_hbm, dinv_hbm, b1_hbm, src_hbm, dst_hbm, s2_hbm,
              src_v, dst_v, row0, row1, p0_v, p1_v, dinv_v, g2_v, b1_v,
              acc, gtab, sem0, sem1):
    c = lax.axis_index("c")
    s = lax.axis_index("s")
    wid = s * 2 + c

    pltpu.sync_copy(src_hbm.at[pl.ds(wid * NB, NB)], src_v)
    pltpu.sync_copy(dst_hbm.at[pl.ds(wid * NB, NB)], dst_v)

    sl = pl.ds(s * RPT, RPT)
    pltpu.sync_copy(s1_hbm.at[0, sl], p0_v)
    pltpu.sync_copy(s1_hbm.at[1, sl], p1_v)
    pltpu.sync_copy(dinv_hbm.at[sl], dinv_v)
    pltpu.sync_copy(b1_hbm, b1_v)

    def scale(i, _):
        dv = dinv_v[i]
        a1 = dv * (p0_v[i] + p1_v[i]) + b1_v[...]
        g2_v[i] = dv * jnp.maximum(a1, 0.0)
        return 0

    lax.fori_loop(0, RPT, scale, 0)

    pltpu.sync_copy(g2_v, gtab.at[sl])

    zero = jnp.zeros((16,), jnp.float32)
    _fill_rows(p0_v, RPT, zero)

    @pl.when(c == 0)
    def _ia():
        pltpu.sync_copy(g2_v, acc.at[sl])

    @pl.when(c != 0)
    def _ib():
        pltpu.sync_copy(p0_v, acc.at[sl])

    plsc.subcore_barrier()
    _edge_pass(gtab, acc, src_v, dst_v, row0, row1, sem0, sem1)
    plsc.subcore_barrier()
    pltpu.sync_copy(acc.at[sl], s2_hbm.at[c, sl])


def _tc_mm1(x_ref, w_ref, o_ref):
    o_ref[:] = jnp.dot(x_ref[:], w_ref[:], preferred_element_type=jnp.float32)


def _tc_mm2(sp_ref, dinv_ref, w_ref, b_ref, o_ref):
    a = dinv_ref[:] * (sp_ref[0] + sp_ref[1])
    o_ref[:] = jnp.dot(a, w_ref[:], preferred_element_type=jnp.float32) + b_ref[:]


def kernel(x, edge_index, W1, b1, W2, b2):
    f32 = jnp.float32

    src2d = edge_index[0].reshape(EROWS, EB)
    dst2d = edge_index[1].reshape(EROWS, EB)

    counts = _sc_count(dst2d)
    h1 = pl.pallas_call(
        _tc_mm1,
        out_shape=jax.ShapeDtypeStruct((N, HID), f32),
    )(x, W1)

    s1, dinv16 = _sc_prop1(h1, counts, src2d, dst2d)
    s2 = _sc_prop2(s1, dinv16, b1, src2d, dst2d)

    out = pl.pallas_call(
        _tc_mm2,
        out_shape=jax.ShapeDtypeStruct((N, D_OUT), f32),
    )(s2, dinv16, W2, b2.reshape(1, D_OUT))

    return out
